# Initial kernel scaffold; baseline (speedup 1.0000x reference)
import jax
import jax.numpy as jnp
from jax.experimental import pallas as pl


def kernel(hidden_states, segment_ids):
    raise NotImplementedError("write your pallas kernel here")


# speedup vs baseline: 1.3085x; 1.3085x over previous
import functools

import jax
import jax.numpy as jnp
from jax import lax
from jax.experimental import pallas as pl
from jax.experimental.pallas import tpu as pltpu
from jax.experimental.pallas import tpu_sc as plsc

N_SEGS = 16
N_TOKENS = 32768
D_MODEL = 1024

NC = 2
NS = 16
NW = NC * NS
F_PER_W = D_MODEL // NW
CHUNK_T = 1024
N_CHUNKS = N_TOKENS // CHUNK_T
IDS_R = 256
IDS_C = 128


def _bounds_tc(ids_ref, out_ref):
    ids = ids_ref[...]
    seg = lax.broadcasted_iota(jnp.int32, (32, IDS_R, IDS_C), 0)
    lt = (ids[None, :, :] < seg).astype(jnp.int32)
    out_ref[...] = jnp.sum(lt, axis=(1, 2)).reshape(1, 32)


def _seg_sum_sc(hid_hbm, bounds_hbm, out_hbm, bounds_v, buf0, buf1, stage,
                in_sem):
    cid = lax.axis_index("c")
    sid = lax.axis_index("s")
    wid = sid * NC + cid
    f0 = wid * F_PER_W

    pltpu.sync_copy(bounds_hbm, bounds_v)
    blo = bounds_v[0, pl.ds(0, 16)]
    bhi = bounds_v[0, pl.ds(16, 16)]
    starts = [blo[s] for s in range(N_SEGS)] + [bhi[0]]

    zeros = jnp.zeros((16,), jnp.float32)
    for r in range(N_SEGS):
        stage[r, pl.ds(0, 16)] = zeros
        stage[r, pl.ds(16, 16)] = zeros

    def _start_in(c, buf):
        return pltpu.async_copy(
            hid_hbm.at[pl.ds(c * CHUNK_T, CHUNK_T), pl.ds(f0, F_PER_W)],
            buf, in_sem)

    def _wait_in(buf):
        pltpu.make_async_copy(
            hid_hbm.at[pl.ds(0, CHUNK_T), pl.ds(f0, F_PER_W)],
            buf, in_sem).wait()

    def _accum_chunk(c, buf):
        base = c * CHUNK_T
        for s in range(N_SEGS):
            t0 = jnp.clip(starts[s] - base, 0, CHUNK_T)
            t1 = jnp.clip(starts[s + 1] - base, 0, CHUNK_T)
            n = t1 - t0
            n8 = (n // 8) * 8

            def body8(i, carry, t0=t0):
                a0, a1 = carry
                b = t0 + i * 8
                for k in range(8):
                    a0 = a0 + buf[b + k, pl.ds(0, 16)]
                    a1 = a1 + buf[b + k, pl.ds(16, 16)]
                return a0, a1

            def body1(t, carry):
                a0, a1 = carry
                a0 = a0 + buf[t, pl.ds(0, 16)]
                a1 = a1 + buf[t, pl.ds(16, 16)]
                return a0, a1

            a0, a1 = lax.fori_loop(0, n8 // 8, body8, (zeros, zeros))
            a0, a1 = lax.fori_loop(t0 + n8, t1, body1, (a0, a1))
            sl0 = (s, pl.ds(0, 16))
            sl1 = (s, pl.ds(16, 16))
            stage[sl0] = stage[sl0] + a0
            stage[sl1] = stage[sl1] + a1

    _start_in(0, buf0)

    def pair_body(cc, carry):
        c0 = cc * 2
        _wait_in(buf0)

        @pl.when(c0 + 1 < N_CHUNKS)
        def _():
            _start_in(c0 + 1, buf1)

        _accum_chunk(c0, buf0)

        @pl.when(c0 + 1 < N_CHUNKS)
        def _():
            _wait_in(buf1)

            @pl.when(c0 + 2 < N_CHUNKS)
            def _():
                _start_in(c0 + 2, buf0)

            _accum_chunk(c0 + 1, buf1)

        return carry

    lax.fori_loop(0, N_CHUNKS // 2, pair_body, 0)

    pltpu.sync_copy(stage, out_hbm.at[:, pl.ds(f0, F_PER_W)])


_seg_sum = functools.partial(
    pl.kernel,
    out_type=jax.ShapeDtypeStruct((N_SEGS, D_MODEL), jnp.float32),
    mesh=plsc.VectorSubcoreMesh(core_axis_name="c", subcore_axis_name="s",
                                num_cores=NC, num_subcores=NS),
    compiler_params=pltpu.CompilerParams(use_tc_tiling_on_sc=False),
    scratch_types=[
        pltpu.VMEM((1, 32), jnp.int32),
        pltpu.VMEM((CHUNK_T, F_PER_W), jnp.float32),
        pltpu.VMEM((CHUNK_T, F_PER_W), jnp.float32),
        pltpu.VMEM((N_SEGS, F_PER_W), jnp.float32),
        pltpu.SemaphoreType.DMA,
    ],
)(_seg_sum_sc)


def _finish_tc(bounds_ref, sums_ref, out_ref):
    bounds = bounds_ref[...]
    sums = sums_ref[...]
    counts = (bounds[0, 1:N_SEGS + 1] - bounds[0, :N_SEGS]).astype(jnp.float32)
    counts = jnp.maximum(counts, 1.0)
    mean = sums / counts[:, None]
    ss = jnp.sum(mean * mean, axis=-1, keepdims=True)
    norm = jnp.maximum(jnp.sqrt(ss), 1e-12)
    out_ref[...] = mean / norm


def kernel(hidden_states, segment_ids):
    ids2d = segment_ids.reshape(IDS_R, IDS_C)
    bounds = pl.pallas_call(
        _bounds_tc,
        out_shape=jax.ShapeDtypeStruct((1, 32), jnp.int32),
    )(ids2d)
    sums = _seg_sum(hidden_states, bounds)
    return pl.pallas_call(
        _finish_tc,
        out_shape=jax.ShapeDtypeStruct((N_SEGS, D_MODEL), jnp.float32),
    )(bounds, sums)


# speedup vs baseline: 2.2506x; 1.7200x over previous
import functools

import jax
import jax.numpy as jnp
from jax import lax
from jax.experimental import pallas as pl
from jax.experimental.pallas import tpu as pltpu
from jax.experimental.pallas import tpu_sc as plsc

N_SEGS = 16
N_TOKENS = 32768
D_MODEL = 1024

NC = 2
NS = 16
NW = NC * NS
N_FG = 8
F_PER_G = D_MODEL // N_FG
N_TG = NW // N_FG
T_PER_G = N_TOKENS // N_TG
CHUNK_T = 256
N_CHUNKS = T_PER_G // CHUNK_T
IDS_R = 256
IDS_C = 128


def _bounds_tc(ids_ref, out_ref):
    ids = ids_ref[...]
    seg = lax.broadcasted_iota(jnp.int32, (32, IDS_R, IDS_C), 0)
    lt = (ids[None, :, :] < seg).astype(jnp.int32)
    cnt = jnp.sum(lt, axis=(1, 2))
    row = jnp.pad(cnt, (0, 96)).reshape(1, 128)
    keep = (lax.broadcasted_iota(jnp.int32, (8, 128), 0) == 0).astype(jnp.int32)
    out_ref[...] = row * keep


def _seg_sum_sc(hid_hbm, bounds_hbm, out_hbm, bounds_v, buf0, buf1, stage,
                in_sem):
    cid = lax.axis_index("c")
    sid = lax.axis_index("s")
    wid = sid * NC + cid
    fg = wid % N_FG
    tg = wid // N_FG
    f0 = fg * F_PER_G
    tok0 = tg * T_PER_G

    pltpu.sync_copy(bounds_hbm, bounds_v)
    blo = bounds_v[0, pl.ds(0, 16)]
    bhi = bounds_v[0, pl.ds(16, 16)]
    starts = [blo[s] for s in range(N_SEGS)] + [bhi[0]]

    zeros = jnp.zeros((16,), jnp.float32)
    for r in range(N_SEGS):
        for g in range(F_PER_G // 16):
            stage[r, pl.ds(g * 16, 16)] = zeros

    def _start_in(c, buf):
        return pltpu.async_copy(
            hid_hbm.at[pl.ds(tok0 + c * CHUNK_T, CHUNK_T), pl.ds(f0, F_PER_G)],
            buf, in_sem)

    def _wait_in(buf):
        pltpu.make_async_copy(
            hid_hbm.at[pl.ds(0, CHUNK_T), pl.ds(f0, F_PER_G)],
            buf, in_sem).wait()

    def _accum_chunk(c, buf):
        base = tok0 + c * CHUNK_T
        for s in range(N_SEGS):
            t0 = jnp.clip(starts[s] - base, 0, CHUNK_T)
            t1 = jnp.clip(starts[s + 1] - base, 0, CHUNK_T)
            n = t1 - t0

            @pl.when(n > 0)
            def _(s=s, t0=t0, t1=t1, n=n):
                n4 = (n // 4) * 4

                def body4(i, carry, t0=t0):
                    acc = list(carry)
                    b = t0 + i * 4
                    for k in range(4):
                        for g in range(8):
                            acc[g] = acc[g] + buf[b + k, pl.ds(g * 16, 16)]
                    return tuple(acc)

                def body1(t, carry):
                    acc = list(carry)
                    for g in range(8):
                        acc[g] = acc[g] + buf[t, pl.ds(g * 16, 16)]
                    return tuple(acc)

                acc = lax.fori_loop(0, n4 // 4, body4, (zeros,) * 8)
                acc = lax.fori_loop(t0 + n4, t1, body1, acc)
                for g in range(8):
                    sl = (s, pl.ds(g * 16, 16))
                    stage[sl] = stage[sl] + acc[g]

    _start_in(0, buf0)

    def pair_body(cc, carry):
        c0 = cc * 2
        _wait_in(buf0)

        @pl.when(c0 + 1 < N_CHUNKS)
        def _():
            _start_in(c0 + 1, buf1)

        _accum_chunk(c0, buf0)

        @pl.when(c0 + 1 < N_CHUNKS)
        def _():
            _wait_in(buf1)

            @pl.when(c0 + 2 < N_CHUNKS)
            def _():
                _start_in(c0 + 2, buf0)

            _accum_chunk(c0 + 1, buf1)

        return carry

    lax.fori_loop(0, N_CHUNKS // 2, pair_body, 0)

    pltpu.sync_copy(stage, out_hbm.at[tg, :, pl.ds(f0, F_PER_G)])


_seg_sum = functools.partial(
    pl.kernel,
    out_type=jax.ShapeDtypeStruct((N_TG, N_SEGS, D_MODEL), jnp.float32),
    mesh=plsc.VectorSubcoreMesh(core_axis_name="c", subcore_axis_name="s",
                                num_cores=NC, num_subcores=NS),
    scratch_types=[
        pltpu.VMEM((8, 128), jnp.int32),
        pltpu.VMEM((CHUNK_T, F_PER_G), jnp.float32),
        pltpu.VMEM((CHUNK_T, F_PER_G), jnp.float32),
        pltpu.VMEM((N_SEGS, F_PER_G), jnp.float32),
        pltpu.SemaphoreType.DMA,
    ],
)(_seg_sum_sc)


def _finish_tc(bounds_ref, part_ref, out_ref):
    bounds = bounds_ref[...]
    part = part_ref[...]
    sums = part[0] + part[1] + part[2] + part[3]
    counts = (bounds[0, 1:N_SEGS + 1] - bounds[0, :N_SEGS]).astype(jnp.float32)
    counts = jnp.maximum(counts, 1.0)
    mean = sums / counts[:, None]
    ss = jnp.sum(mean * mean, axis=-1, keepdims=True)
    norm = jnp.maximum(jnp.sqrt(ss), 1e-12)
    out_ref[...] = mean / norm


def kernel(hidden_states, segment_ids):
    ids2d = segment_ids.reshape(IDS_R, IDS_C)
    bounds = pl.pallas_call(
        _bounds_tc,
        out_shape=jax.ShapeDtypeStruct((8, 128), jnp.int32),
    )(ids2d)
    part = _seg_sum(hidden_states, bounds)
    return pl.pallas_call(
        _finish_tc,
        out_shape=jax.ShapeDtypeStruct((N_SEGS, D_MODEL), jnp.float32),
    )(bounds, part)


# speedup vs baseline: 2.9672x; 1.3184x over previous
import functools

import jax
import jax.numpy as jnp
from jax import lax
from jax.experimental import pallas as pl
from jax.experimental.pallas import tpu as pltpu
from jax.experimental.pallas import tpu_sc as plsc

N_SEGS = 16
N_TOKENS = 32768
D_MODEL = 1024

NC = 2
NS = 16
NW = NC * NS
N_FG = 8
F_PER_G = D_MODEL // N_FG
N_TG = NW // N_FG
T_PER_G = N_TOKENS // N_TG
CHUNK_T = 256
N_CHUNKS = T_PER_G // CHUNK_T
IDS_R = 256
IDS_C = 128


def _bounds_tc(ids_ref, out_ref):
    ids = ids_ref[...]
    seg = lax.broadcasted_iota(jnp.int32, (32, IDS_R, IDS_C), 0)
    lt = (ids[None, :, :] < seg).astype(jnp.int32)
    cnt = jnp.sum(lt, axis=(1, 2))
    row = jnp.pad(cnt, (0, 96)).reshape(1, 128)
    keep = (lax.broadcasted_iota(jnp.int32, (8, 128), 0) == 0).astype(jnp.int32)
    out_ref[...] = row * keep


def _seg_sum_sc(hid_hbm, bounds_hbm, out_hbm, bounds_v, buf0, buf1, stage,
                in_sem):
    cid = lax.axis_index("c")
    sid = lax.axis_index("s")
    wid = sid * NC + cid
    fg = wid % N_FG
    tg = wid // N_FG
    f0 = fg * F_PER_G
    tok0 = tg * T_PER_G

    pltpu.sync_copy(bounds_hbm, bounds_v)
    blo = bounds_v[0, pl.ds(0, 16)]
    bhi = bounds_v[0, pl.ds(16, 16)]
    starts = [blo[s] for s in range(N_SEGS)] + [bhi[0]]

    zeros = jnp.zeros((16,), jnp.float32)
    for r in range(N_SEGS):
        for g in range(F_PER_G // 16):
            stage[r, pl.ds(g * 16, 16)] = zeros

    def _start_in(c, buf):
        return pltpu.async_copy(
            hid_hbm.at[pl.ds(tok0 + c * CHUNK_T, CHUNK_T), pl.ds(f0, F_PER_G)],
            buf, in_sem)

    def _wait_in(buf):
        pltpu.make_async_copy(
            hid_hbm.at[pl.ds(0, CHUNK_T), pl.ds(f0, F_PER_G)],
            buf, in_sem).wait()

    def _accum_chunk(c, buf):
        base = tok0 + c * CHUNK_T
        sfirst = -1
        slast = -1
        for s in range(N_SEGS):
            sfirst = sfirst + jnp.where(starts[s] <= base, 1, 0)
            slast = slast + jnp.where(starts[s] <= base + CHUNK_T - 1, 1, 0)

        @pl.when(sfirst == slast)
        def _():
            def body8(i, carry):
                acc = list(carry)
                b = i * 8
                for k in range(8):
                    for g in range(8):
                        acc[g] = acc[g] + buf[b + k, pl.ds(g * 16, 16)]
                return tuple(acc)

            acc = lax.fori_loop(0, CHUNK_T // 8, body8, (zeros,) * 8)
            for g in range(8):
                sl = (sfirst, pl.ds(g * 16, 16))
                stage[sl] = stage[sl] + acc[g]

        @pl.when(sfirst != slast)
        def _():
            _accum_chunk_mixed(base, buf)

    def _accum_chunk_mixed(base, buf):
        for s in range(N_SEGS):
            t0 = jnp.clip(starts[s] - base, 0, CHUNK_T)
            t1 = jnp.clip(starts[s + 1] - base, 0, CHUNK_T)
            n = t1 - t0

            @pl.when(n > 0)
            def _(s=s, t0=t0, t1=t1, n=n):
                n4 = (n // 4) * 4

                def body4(i, carry, t0=t0):
                    acc = list(carry)
                    b = t0 + i * 4
                    for k in range(4):
                        for g in range(8):
                            acc[g] = acc[g] + buf[b + k, pl.ds(g * 16, 16)]
                    return tuple(acc)

                def body1(t, carry):
                    acc = list(carry)
                    for g in range(8):
                        acc[g] = acc[g] + buf[t, pl.ds(g * 16, 16)]
                    return tuple(acc)

                acc = lax.fori_loop(0, n4 // 4, body4, (zeros,) * 8)
                acc = lax.fori_loop(t0 + n4, t1, body1, acc)
                for g in range(8):
                    sl = (s, pl.ds(g * 16, 16))
                    stage[sl] = stage[sl] + acc[g]

    _start_in(0, buf0)

    def pair_body(cc, carry):
        c0 = cc * 2
        _wait_in(buf0)

        @pl.when(c0 + 1 < N_CHUNKS)
        def _():
            _start_in(c0 + 1, buf1)

        _accum_chunk(c0, buf0)

        @pl.when(c0 + 1 < N_CHUNKS)
        def _():
            _wait_in(buf1)

            @pl.when(c0 + 2 < N_CHUNKS)
            def _():
                _start_in(c0 + 2, buf0)

            _accum_chunk(c0 + 1, buf1)

        return carry

    lax.fori_loop(0, N_CHUNKS // 2, pair_body, 0)

    pltpu.sync_copy(stage, out_hbm.at[tg, :, pl.ds(f0, F_PER_G)])


_seg_sum = functools.partial(
    pl.kernel,
    out_type=jax.ShapeDtypeStruct((N_TG, N_SEGS, D_MODEL), jnp.float32),
    mesh=plsc.VectorSubcoreMesh(core_axis_name="c", subcore_axis_name="s",
                                num_cores=NC, num_subcores=NS),
    scratch_types=[
        pltpu.VMEM((8, 128), jnp.int32),
        pltpu.VMEM((CHUNK_T, F_PER_G), jnp.float32),
        pltpu.VMEM((CHUNK_T, F_PER_G), jnp.float32),
        pltpu.VMEM((N_SEGS, F_PER_G), jnp.float32),
        pltpu.SemaphoreType.DMA,
    ],
)(_seg_sum_sc)


def _finish_tc(bounds_ref, part_ref, out_ref):
    bounds = bounds_ref[...]
    part = part_ref[...]
    sums = part[0] + part[1] + part[2] + part[3]
    counts = (bounds[0, 1:N_SEGS + 1] - bounds[0, :N_SEGS]).astype(jnp.float32)
    counts = jnp.maximum(counts, 1.0)
    mean = sums / counts[:, None]
    ss = jnp.sum(mean * mean, axis=-1, keepdims=True)
    norm = jnp.maximum(jnp.sqrt(ss), 1e-12)
    out_ref[...] = mean / norm


def kernel(hidden_states, segment_ids):
    ids2d = segment_ids.reshape(IDS_R, IDS_C)
    bounds = pl.pallas_call(
        _bounds_tc,
        out_shape=jax.ShapeDtypeStruct((8, 128), jnp.int32),
    )(ids2d)
    part = _seg_sum(hidden_states, bounds)
    return pl.pallas_call(
        _finish_tc,
        out_shape=jax.ShapeDtypeStruct((N_SEGS, D_MODEL), jnp.float32),
    )(bounds, part)
